# baseline (speedup 1.0000x reference)
import functools

import jax
import jax.numpy as jnp
from jax.experimental import pallas as pl
from jax.experimental.pallas import tpu as pltpu

_T = 256
_RT = 512


def _router_body(x_ref, wg_ref, bg_ref, g_ref, eid_ref, gate_ref):
    z = jnp.dot(x_ref[...], wg_ref[...], preferred_element_type=jnp.float32)
    z = z + bg_ref[...] + g_ref[...]
    m = jnp.max(z, axis=1, keepdims=True)
    e = jnp.exp(z - m)
    ssum = jnp.sum(e, axis=1, keepdims=True)
    y = e / ssum
    my = jnp.max(y, axis=1, keepdims=True)
    iota = jax.lax.broadcasted_iota(jnp.int32, z.shape, 1)
    idx = jnp.min(jnp.where(y == my, iota, z.shape[1]), axis=1)
    s = jnp.max(y, axis=1)
    gate = (1.0 + s) - s
    eid_ref[...] = idx[:, None].astype(jnp.int32)
    gate_ref[...] = gate[:, None]


def _expert_body(te_ref, xs_ref, w1_ref, b1_ref, w2_ref, b2_ref, gs_ref,
                 out_ref):
    xb = xs_ref[...]
    h = jnp.dot(xb, w1_ref[0], preferred_element_type=jnp.float32)
    h = jnp.maximum(h + b1_ref[0], 0.0)
    o = jnp.dot(h, w2_ref[0], preferred_element_type=jnp.float32)
    o = o + b2_ref[0, 0, 0]
    out_ref[...] = o * gs_ref[...]


def kernel(x, Wg, bg, W1, b1, W2, b2):
    B, S, D = x.shape
    N = B * S
    E = W1.shape[0]
    x_flat = x.reshape(N, D)

    u = jax.random.uniform(jax.random.key(42), (N, E), minval=1e-20,
                           maxval=1.0, dtype=x.dtype)
    g = -jnp.log(-jnp.log(u))

    eid2, gate2 = pl.pallas_call(
        _router_body,
        grid=(N // _RT,),
        in_specs=[
            pl.BlockSpec((_RT, D), lambda i: (i, 0)),
            pl.BlockSpec((D, E), lambda i: (0, 0)),
            pl.BlockSpec((1, E), lambda i: (0, 0)),
            pl.BlockSpec((_RT, E), lambda i: (i, 0)),
        ],
        out_specs=[
            pl.BlockSpec((_RT, 1), lambda i: (i, 0)),
            pl.BlockSpec((_RT, 1), lambda i: (i, 0)),
        ],
        out_shape=[
            jax.ShapeDtypeStruct((N, 1), jnp.int32),
            jax.ShapeDtypeStruct((N, 1), jnp.float32),
        ],
    )(x_flat, Wg, bg.reshape(1, E), g)
    eid = eid2[:, 0]
    gate = gate2[:, 0]

    G = N // _T + E
    sort_idx = jnp.argsort(eid)
    e_sorted = eid[sort_idx]
    counts = jnp.sum(eid[:, None] == jnp.arange(E)[None, :], axis=0,
                     dtype=jnp.int32)
    tiles_e = (counts + _T - 1) // _T
    cum_tiles = jnp.cumsum(tiles_e)
    tile_start = jnp.concatenate(
        [jnp.zeros((1,), jnp.int32), cum_tiles[:-1]])
    offsets = jnp.concatenate(
        [jnp.zeros((1,), jnp.int32), jnp.cumsum(counts)[:-1]])
    r = jnp.arange(N, dtype=jnp.int32)
    padded_pos = tile_start[e_sorted] * _T + (r - offsets[e_sorted])
    src = jnp.zeros((G * _T,), jnp.int32).at[padded_pos].set(sort_idx)
    gate_pad = jnp.zeros((G * _T,), jnp.float32).at[padded_pos].set(
        gate[sort_idx])
    tvec = jnp.arange(G, dtype=jnp.int32)
    tile_expert = jnp.minimum(
        jnp.sum(tvec[:, None] >= cum_tiles[None, :], axis=1),
        E - 1).astype(jnp.int32)

    x_sorted = x_flat[src]

    grid_spec = pltpu.PrefetchScalarGridSpec(
        num_scalar_prefetch=1,
        grid=(G,),
        in_specs=[
            pl.BlockSpec((_T, D), lambda i, te: (i, 0)),
            pl.BlockSpec((1, D, D), lambda i, te: (te[i], 0, 0)),
            pl.BlockSpec((1, 1, D), lambda i, te: (te[i], 0, 0)),
            pl.BlockSpec((1, D, 1), lambda i, te: (te[i], 0, 0)),
            pl.BlockSpec((1, 1, 1), lambda i, te: (te[i], 0, 0)),
            pl.BlockSpec((_T, 1), lambda i, te: (i, 0)),
        ],
        out_specs=pl.BlockSpec((_T, 1), lambda i, te: (i, 0)),
    )
    out_sorted = pl.pallas_call(
        _expert_body,
        grid_spec=grid_spec,
        out_shape=jax.ShapeDtypeStruct((G * _T, 1), jnp.float32),
    )(tile_expert, x_sorted, W1, b1.reshape(E, 1, D), W2,
      b2.reshape(E, 1, 1), gate_pad[:, None])

    token_pos = jnp.zeros((N,), jnp.int32).at[sort_idx].set(padded_pos)
    out = out_sorted[token_pos, 0]
    return out.reshape(B, S, 1)


# speedup vs baseline: 1.8595x; 1.8595x over previous
import functools

import jax
import jax.numpy as jnp
from jax import lax
from jax.experimental import pallas as pl
from jax.experimental.pallas import tpu as pltpu
from jax.experimental.pallas import tpu_sc as plsc

_T = 256
_RT = 512
_NW = 32
_BATCH = 32


def _router_body(x_ref, wg_ref, bg_ref, g_ref, eid_ref, gate_ref, rank_ref,
                 counts_ref, cnt):
    i = pl.program_id(0)
    E = wg_ref.shape[1]
    z = jnp.dot(x_ref[...], wg_ref[...], preferred_element_type=jnp.float32)
    z = z + bg_ref[...] + g_ref[...]
    m = jnp.max(z, axis=1, keepdims=True)
    e = jnp.exp(z - m)
    ssum = jnp.sum(e, axis=1, keepdims=True)
    y = e / ssum
    my = jnp.max(y, axis=1, keepdims=True)
    iota = lax.broadcasted_iota(jnp.int32, z.shape, 1)
    idx = jnp.min(jnp.where(y == my, iota, E), axis=1)
    s = jnp.max(y, axis=1)
    gate = (1.0 + s) - s

    onehot = (idx[:, None] == iota).astype(jnp.float32)
    rows = lax.broadcasted_iota(jnp.int32, (z.shape[0], z.shape[0]), 0)
    cols = lax.broadcasted_iota(jnp.int32, (z.shape[0], z.shape[0]), 1)
    tril = (cols <= rows).astype(jnp.float32)
    rank_incl = jnp.dot(tril, onehot, preferred_element_type=jnp.float32)
    pos_in_blk = jnp.sum(rank_incl * onehot, axis=1) - 1.0

    @pl.when(i == 0)
    def _():
        cnt[...] = jnp.zeros_like(cnt)

    base = cnt[...]
    grank = pos_in_blk + jnp.sum(onehot * base, axis=1)
    newc = base + jnp.sum(onehot, axis=0, keepdims=True)
    cnt[...] = newc
    counts_ref[...] = newc

    eid_ref[...] = idx[:, None]
    gate_ref[...] = gate[:, None]
    rank_ref[...] = grank.astype(jnp.int32)[:, None]


def _expert_body(te_ref, xs_ref, w1_ref, b1_ref, w2_ref, b2_ref, out_ref):
    h = jnp.dot(xs_ref[...], w1_ref[0], preferred_element_type=jnp.float32)
    h = jnp.maximum(h + b1_ref[0], 0.0)
    o = jnp.dot(h, w2_ref[0], preferred_element_type=jnp.float32)
    out_ref[...] = o + b2_ref[0, 0, 0]


def _make_sc_scatter(N, D, GT):
    rows_per_w = N // _NW
    K = rows_per_w // _BATCH
    mesh = plsc.VectorSubcoreMesh(core_axis_name="c", subcore_axis_name="s")

    @functools.partial(
        pl.kernel,
        out_type=jax.ShapeDtypeStruct((GT, D), jnp.float32),
        mesh=mesh,
        scratch_types=[
            pltpu.VMEM((K, _BATCH), jnp.int32),
            pltpu.VMEM((_BATCH, D), jnp.float32),
            pltpu.VMEM((_BATCH, D), jnp.float32),
            pltpu.SemaphoreType.DMA,
            pltpu.SemaphoreType.DMA,
        ],
    )
    def sc_scatter(x_hbm, pos_hbm, out_hbm, idx_v, buf0, buf1, sem_ld,
                   sem_st):
        w = lax.axis_index("s") * 2 + lax.axis_index("c")
        base = w * rows_per_w
        pltpu.sync_copy(pos_hbm.at[w], idx_v)
        bufs = (buf0, buf1)
        loads = [None, None]
        scats = [None, None]
        loads[0] = pltpu.async_copy(
            x_hbm.at[pl.ds(base, _BATCH)], bufs[0], sem_ld)
        for j in range(K):
            loads[j % 2].wait()
            if j + 1 < K:
                if scats[(j + 1) % 2] is not None:
                    scats[(j + 1) % 2].wait()
                loads[(j + 1) % 2] = pltpu.async_copy(
                    x_hbm.at[pl.ds(base + (j + 1) * _BATCH, _BATCH)],
                    bufs[(j + 1) % 2], sem_ld)
            scats[j % 2] = pltpu.async_copy(
                bufs[j % 2], out_hbm.at[idx_v.at[j]], sem_st)
        scats[(K - 2) % 2].wait()
        scats[(K - 1) % 2].wait()

    return sc_scatter


def kernel(x, Wg, bg, W1, b1, W2, b2):
    B, S, D = x.shape
    N = B * S
    E = W1.shape[0]
    x_flat = x.reshape(N, D)

    u = jax.random.uniform(jax.random.key(42), (N, E), minval=1e-20,
                           maxval=1.0, dtype=x.dtype)
    g = -jnp.log(-jnp.log(u))

    eid2, gate2, rank2, counts_f = pl.pallas_call(
        _router_body,
        grid=(N // _RT,),
        in_specs=[
            pl.BlockSpec((_RT, D), lambda i: (i, 0)),
            pl.BlockSpec((D, E), lambda i: (0, 0)),
            pl.BlockSpec((1, E), lambda i: (0, 0)),
            pl.BlockSpec((_RT, E), lambda i: (i, 0)),
        ],
        out_specs=[
            pl.BlockSpec((_RT, 1), lambda i: (i, 0)),
            pl.BlockSpec((_RT, 1), lambda i: (i, 0)),
            pl.BlockSpec((_RT, 1), lambda i: (i, 0)),
            pl.BlockSpec((1, E), lambda i: (0, 0)),
        ],
        out_shape=[
            jax.ShapeDtypeStruct((N, 1), jnp.int32),
            jax.ShapeDtypeStruct((N, 1), jnp.float32),
            jax.ShapeDtypeStruct((N, 1), jnp.int32),
            jax.ShapeDtypeStruct((1, E), jnp.float32),
        ],
        scratch_shapes=[pltpu.VMEM((1, E), jnp.float32)],
    )(x_flat, Wg, bg.reshape(1, E), g)
    eid = eid2[:, 0]
    gate = gate2[:, 0]
    rank = rank2[:, 0]
    counts = counts_f[0].astype(jnp.int32)

    G = N // _T + E
    tiles_e = (counts + _T - 1) // _T
    cum_tiles = jnp.cumsum(tiles_e)
    tile_start = jnp.concatenate(
        [jnp.zeros((1,), jnp.int32), cum_tiles[:-1]])
    tvec = jnp.arange(G, dtype=jnp.int32)
    tile_expert = jnp.minimum(
        jnp.sum(tvec[:, None] >= cum_tiles[None, :], axis=1),
        E - 1).astype(jnp.int32)
    padded_pos = tile_start[eid] * _T + rank

    x_sorted = _make_sc_scatter(N, D, G * _T)(
        x_flat, padded_pos.reshape(_NW, N // _NW // _BATCH, _BATCH))

    grid_spec = pltpu.PrefetchScalarGridSpec(
        num_scalar_prefetch=1,
        grid=(G,),
        in_specs=[
            pl.BlockSpec((_T, D), lambda i, te: (i, 0)),
            pl.BlockSpec((1, D, D), lambda i, te: (te[i], 0, 0)),
            pl.BlockSpec((1, 1, D), lambda i, te: (te[i], 0, 0)),
            pl.BlockSpec((1, D, 1), lambda i, te: (te[i], 0, 0)),
            pl.BlockSpec((1, 1, 1), lambda i, te: (te[i], 0, 0)),
        ],
        out_specs=pl.BlockSpec((_T, 1), lambda i, te: (i, 0)),
    )
    out_sorted = pl.pallas_call(
        _expert_body,
        grid_spec=grid_spec,
        out_shape=jax.ShapeDtypeStruct((G * _T, 1), jnp.float32),
    )(tile_expert, x_sorted, W1, b1.reshape(E, 1, D), W2,
      b2.reshape(E, 1, 1))

    out = out_sorted[padded_pos, 0] * gate
    return out.reshape(B, S, 1)
